# Initial kernel scaffold; baseline (speedup 1.0000x reference)
import jax
import jax.numpy as jnp
from jax.experimental import pallas as pl


def kernel(x, edge_index, W1, b1, W2, b2, W3, b3):
    raise NotImplementedError("write your pallas kernel here")


# speedup vs baseline: 17.7883x; 17.7883x over previous
import functools

import jax
import jax.numpy as jnp
from jax import lax
from jax.experimental import pallas as pl
from jax.experimental.pallas import tpu as pltpu
from jax.experimental.pallas import tpu_sc as plsc

_N = 10000
_E = 320000
_D = 128
_DO = 128
_NCLS = 40

_NTILES = 32
_K = 128
_PT = 10240
_EP = _NTILES * _PT
_CHUNKS = _PT // _K
_ACC_N = 10112
_RPT = _ACC_N // 16
_BN = 1000


def _sc_mesh():
    return plsc.VectorSubcoreMesh(core_axis_name="c", subcore_axis_name="s")


def _make_deg_kernel():
    @functools.partial(
        pl.kernel,
        mesh=_sc_mesh(),
        out_type=jax.ShapeDtypeStruct((2, _ACC_N, _D), jnp.float32),
        scratch_types=[
            pltpu.VMEM((_CHUNKS, _K), jnp.int32),
            pltpu.VMEM((_K, _D), jnp.float32),
            pltpu.VMEM_SHARED((_ACC_N, _D), jnp.float32),
        ],
    )
    def deg_kernel(dst_hbm, ones_hbm, zero_hbm, out_hbm, dstv, onesv, acc):
        c = lax.axis_index("c")
        s = lax.axis_index("s")
        t = s * 2 + c
        row0 = s * _RPT
        pltpu.sync_copy(zero_hbm, acc.at[pl.ds(row0, _RPT)])
        pltpu.sync_copy(ones_hbm, onesv)
        pltpu.sync_copy(dst_hbm.at[pl.ds(t * _CHUNKS, _CHUNKS)], dstv)
        plsc.subcore_barrier()

        def body(j, carry):
            pltpu.sync_copy(onesv, acc.at[dstv.at[j]], add=True)
            return carry

        lax.fori_loop(0, _CHUNKS, body, 0)
        plsc.subcore_barrier()
        pltpu.sync_copy(acc.at[pl.ds(row0, _RPT)],
                        out_hbm.at[c, pl.ds(row0, _RPT)])

    return deg_kernel


def _make_scatter_kernel(d):
    @functools.partial(
        pl.kernel,
        mesh=_sc_mesh(),
        out_type=jax.ShapeDtypeStruct((2, _ACC_N, d), jnp.float32),
        scratch_types=[
            pltpu.VMEM((_CHUNKS, _K), jnp.int32),
            pltpu.VMEM((_CHUNKS, _K), jnp.int32),
            pltpu.VMEM((_K, d), jnp.float32),
            pltpu.VMEM_SHARED((_ACC_N, d), jnp.float32),
            pltpu.SemaphoreType.DMA,
        ],
    )
    def scat_kernel(hp_hbm, src_hbm, dst_hbm, zero_hbm, out_hbm,
                    srcv, dstv, rows, acc, sem):
        c = lax.axis_index("c")
        s = lax.axis_index("s")
        t = s * 2 + c
        row0 = s * _RPT
        pltpu.sync_copy(zero_hbm, acc.at[pl.ds(row0, _RPT)])
        pltpu.sync_copy(src_hbm.at[pl.ds(t * _CHUNKS, _CHUNKS)], srcv)
        pltpu.sync_copy(dst_hbm.at[pl.ds(t * _CHUNKS, _CHUNKS)], dstv)
        plsc.subcore_barrier()

        def body(j, carry):
            pltpu.async_copy(hp_hbm.at[srcv.at[j]], rows, sem).wait()
            pltpu.sync_copy(rows, acc.at[dstv.at[j]], add=True)
            return carry

        lax.fori_loop(0, _CHUNKS, body, 0)
        plsc.subcore_barrier()
        pltpu.sync_copy(acc.at[pl.ds(row0, _RPT)],
                        out_hbm.at[c, pl.ds(row0, _RPT)])

    return scat_kernel


def _deg_s(dp0, dp1):
    deg = dp0[:, 0:1] + dp1[:, 0:1] + 1.0
    return lax.rsqrt(deg)


def _t_first(x, w1, degp):
    def body(x_ref, w_ref, dp_ref, o_ref):
        s = _deg_s(dp_ref[0], dp_ref[1])
        h = jnp.dot(x_ref[...], w_ref[...], preferred_element_type=jnp.float32)
        o_ref[...] = h * s

    return pl.pallas_call(
        body,
        grid=(_N // _BN,),
        in_specs=[
            pl.BlockSpec((_BN, _D), lambda i: (i, 0)),
            pl.BlockSpec((_D, _D), lambda i: (0, 0)),
            pl.BlockSpec((2, _BN, _D), lambda i: (0, i, 0)),
        ],
        out_specs=pl.BlockSpec((_BN, _D), lambda i: (i, 0)),
        out_shape=jax.ShapeDtypeStruct((_N, _D), jnp.float32),
    )(x, w1, degp)


def _t_mid(aggp, hp, degp, b_row, w, dout):
    def body(ap_ref, hp_ref, dp_ref, b_ref, w_ref, o_ref):
        s = _deg_s(dp_ref[0], dp_ref[1])
        h = (ap_ref[0] + ap_ref[1] + hp_ref[...]) * s + b_ref[...]
        h = jnp.maximum(h, 0.0)
        o_ref[...] = jnp.dot(h, w_ref[...],
                             preferred_element_type=jnp.float32) * s

    return pl.pallas_call(
        body,
        grid=(_N // _BN,),
        in_specs=[
            pl.BlockSpec((2, _BN, _D), lambda i: (0, i, 0)),
            pl.BlockSpec((_BN, _D), lambda i: (i, 0)),
            pl.BlockSpec((2, _BN, _D), lambda i: (0, i, 0)),
            pl.BlockSpec((1, _D), lambda i: (0, 0)),
            pl.BlockSpec((_D, dout), lambda i: (0, 0)),
        ],
        out_specs=pl.BlockSpec((_BN, dout), lambda i: (i, 0)),
        out_shape=jax.ShapeDtypeStruct((_N, dout), jnp.float32),
    )(aggp, hp, degp, b_row, w)


def _t_last(aggp, hp, degp, b_row):
    def body(ap_ref, hp_ref, dp_ref, b_ref, o_ref):
        s = _deg_s(dp_ref[0], dp_ref[1])
        o = (ap_ref[0] + ap_ref[1] + hp_ref[...]) * s + b_ref[...]
        col = lax.broadcasted_iota(jnp.int32, (_BN, _DO), 1)
        mask = col < _NCLS
        om = jnp.where(mask, o, -jnp.inf)
        m = jnp.max(om, axis=1, keepdims=True)
        e = jnp.where(mask, jnp.exp(o - m), 0.0)
        lse = jnp.log(jnp.sum(e, axis=1, keepdims=True))
        r = o - m - lse
        o_ref[...] = r[:, :_NCLS]

    return pl.pallas_call(
        body,
        grid=(_N // _BN,),
        in_specs=[
            pl.BlockSpec((2, _BN, _DO), lambda i: (0, i, 0)),
            pl.BlockSpec((_BN, _DO), lambda i: (i, 0)),
            pl.BlockSpec((2, _BN, _D), lambda i: (0, i, 0)),
            pl.BlockSpec((1, _DO), lambda i: (0, 0)),
        ],
        out_specs=pl.BlockSpec((_BN, _NCLS), lambda i: (i, 0)),
        out_shape=jax.ShapeDtypeStruct((_N, _NCLS), jnp.float32),
    )(aggp, hp, degp, b_row)


def kernel(x, edge_index, W1, b1, W2, b2, W3, b3):
    src = edge_index[0].astype(jnp.int32)
    dst = edge_index[1].astype(jnp.int32)

    npad = _EP - _E
    pad_i = jnp.arange(npad, dtype=jnp.int32)
    src_p = jnp.concatenate([src, pad_i % _N]).reshape(_EP // _K, _K)
    dst_p = jnp.concatenate([dst, _N + (pad_i % (_ACC_N - _N))]).reshape(
        _EP // _K, _K)

    ones128 = jnp.zeros((_K, _D), jnp.float32).at[:, 0].set(1.0)
    z128 = jnp.zeros((_RPT, _D), jnp.float32)

    b1r = b1.reshape(1, _D)
    b2r = b2.reshape(1, _D)
    b3r = jnp.concatenate([b3, jnp.zeros((_DO - _NCLS,), jnp.float32)]
                          ).reshape(1, _DO)
    w3p = jnp.concatenate(
        [W3, jnp.zeros((_D, _DO - _NCLS), jnp.float32)], axis=1)

    degp = _make_deg_kernel()(dst_p, ones128, z128)

    hp1 = _t_first(x, W1, degp)
    agg1 = _make_scatter_kernel(_D)(hp1, src_p, dst_p, z128)
    hp2 = _t_mid(agg1, hp1, degp, b1r, W2, _D)
    agg2 = _make_scatter_kernel(_D)(hp2, src_p, dst_p, z128)
    hp3 = _t_mid(agg2, hp2, degp, b2r, w3p, _DO)
    agg3 = _make_scatter_kernel(_DO)(hp3, src_p, dst_p, z128)
    return _t_last(agg3, hp3, degp, b3r)


# speedup vs baseline: 24.7589x; 1.3919x over previous
import functools

import jax
import jax.numpy as jnp
from jax import lax
from jax.experimental import pallas as pl
from jax.experimental.pallas import tpu as pltpu
from jax.experimental.pallas import tpu_sc as plsc

_N = 10000
_E = 320000
_D = 128
_DO = 128
_NCLS = 40

_NTILES = 32
_K = 128
_PT = 10240
_EP = _NTILES * _PT
_CHUNKS = _PT // _K
_ACC_N = 10112
_RPT = _ACC_N // 16
_BN = 1000


def _sc_mesh():
    return plsc.VectorSubcoreMesh(core_axis_name="c", subcore_axis_name="s")


def _make_deg_kernel():
    @functools.partial(
        pl.kernel,
        mesh=_sc_mesh(),
        out_type=jax.ShapeDtypeStruct((2, _ACC_N, _D), jnp.float32),
        scratch_types=[
            pltpu.VMEM((_CHUNKS, _K), jnp.int32),
            pltpu.VMEM((_K, _D), jnp.float32),
            pltpu.VMEM_SHARED((_ACC_N, _D), jnp.float32),
        ],
    )
    def deg_kernel(dst_hbm, ones_hbm, zero_hbm, out_hbm, dstv, onesv, acc):
        c = lax.axis_index("c")
        s = lax.axis_index("s")
        t = s * 2 + c
        row0 = s * _RPT
        pltpu.sync_copy(zero_hbm, acc.at[pl.ds(row0, _RPT)])
        pltpu.sync_copy(ones_hbm, onesv)
        pltpu.sync_copy(dst_hbm.at[pl.ds(t * _CHUNKS, _CHUNKS)], dstv)
        plsc.subcore_barrier()

        def body(j, carry):
            pltpu.sync_copy(onesv, acc.at[dstv.at[j]], add=True)
            return carry

        lax.fori_loop(0, _CHUNKS, body, 0)
        plsc.subcore_barrier()
        pltpu.sync_copy(acc.at[pl.ds(row0, _RPT)],
                        out_hbm.at[c, pl.ds(row0, _RPT)])

    return deg_kernel


def _make_scatter_kernel(d):
    ph_chunks = _CHUNKS // 2

    @functools.partial(
        pl.kernel,
        mesh=_sc_mesh(),
        out_type=jax.ShapeDtypeStruct((2, _ACC_N, d), jnp.float32),
        scratch_types=[
            pltpu.VMEM((ph_chunks, _K), jnp.int32),
            pltpu.VMEM((ph_chunks, _K), jnp.int32),
            pltpu.VMEM((_K, d), jnp.float32),
            pltpu.VMEM((_K, d), jnp.float32),
            pltpu.VMEM_SHARED((_ACC_N, d), jnp.float32),
            pltpu.SemaphoreType.DMA,
            pltpu.SemaphoreType.DMA,
        ],
    )
    def scat_kernel(hp_hbm, src_hbm, dst_hbm, zero_hbm, out_hbm,
                    srcv, dstv, rows0, rows1, acc, sem0, sem1):
        c = lax.axis_index("c")
        s = lax.axis_index("s")
        t = s * 2 + c
        row0 = s * _RPT
        pltpu.sync_copy(zero_hbm, acc.at[pl.ds(row0, _RPT)])
        plsc.subcore_barrier()

        def _wait(buf, sem):
            pltpu.make_async_copy(hp_hbm.at[pl.ds(0, _K)], buf, sem).wait()

        for phase in range(2):
            base = t * _CHUNKS + phase * ph_chunks
            pltpu.sync_copy(src_hbm.at[pl.ds(base, ph_chunks)], srcv)
            pltpu.sync_copy(dst_hbm.at[pl.ds(base, ph_chunks)], dstv)

            pltpu.async_copy(hp_hbm.at[srcv.at[0]], rows0, sem0)

            def body(i, carry):
                j0 = 2 * i
                j1 = j0 + 1
                pltpu.async_copy(hp_hbm.at[srcv.at[j1]], rows1, sem1)
                _wait(rows0, sem0)
                pltpu.sync_copy(rows0, acc.at[dstv.at[j0]], add=True)
                jn = jnp.minimum(j1 + 1, ph_chunks - 1)
                pltpu.async_copy(hp_hbm.at[srcv.at[jn]], rows0, sem0)
                _wait(rows1, sem1)
                pltpu.sync_copy(rows1, acc.at[dstv.at[j1]], add=True)
                return carry

            lax.fori_loop(0, ph_chunks // 2, body, 0)
            _wait(rows0, sem0)
        plsc.subcore_barrier()
        pltpu.sync_copy(acc.at[pl.ds(row0, _RPT)],
                        out_hbm.at[c, pl.ds(row0, _RPT)])

    return scat_kernel


def _deg_s(dp0, dp1):
    deg = dp0[:, 0:1] + dp1[:, 0:1] + 1.0
    return lax.rsqrt(deg)


def _t_first(x, w1, degp):
    def body(x_ref, w_ref, dp_ref, o_ref):
        s = _deg_s(dp_ref[0], dp_ref[1])
        h = jnp.dot(x_ref[...], w_ref[...], preferred_element_type=jnp.float32)
        o_ref[...] = h * s

    return pl.pallas_call(
        body,
        grid=(_N // _BN,),
        in_specs=[
            pl.BlockSpec((_BN, _D), lambda i: (i, 0)),
            pl.BlockSpec((_D, _D), lambda i: (0, 0)),
            pl.BlockSpec((2, _BN, _D), lambda i: (0, i, 0)),
        ],
        out_specs=pl.BlockSpec((_BN, _D), lambda i: (i, 0)),
        out_shape=jax.ShapeDtypeStruct((_N, _D), jnp.float32),
    )(x, w1, degp)


def _t_mid(aggp, hp, degp, b_row, w, dout):
    def body(ap_ref, hp_ref, dp_ref, b_ref, w_ref, o_ref):
        s = _deg_s(dp_ref[0], dp_ref[1])
        h = (ap_ref[0] + ap_ref[1] + hp_ref[...]) * s + b_ref[...]
        h = jnp.maximum(h, 0.0)
        o_ref[...] = jnp.dot(h, w_ref[...],
                             preferred_element_type=jnp.float32) * s

    return pl.pallas_call(
        body,
        grid=(_N // _BN,),
        in_specs=[
            pl.BlockSpec((2, _BN, _D), lambda i: (0, i, 0)),
            pl.BlockSpec((_BN, _D), lambda i: (i, 0)),
            pl.BlockSpec((2, _BN, _D), lambda i: (0, i, 0)),
            pl.BlockSpec((1, _D), lambda i: (0, 0)),
            pl.BlockSpec((_D, dout), lambda i: (0, 0)),
        ],
        out_specs=pl.BlockSpec((_BN, dout), lambda i: (i, 0)),
        out_shape=jax.ShapeDtypeStruct((_N, dout), jnp.float32),
    )(aggp, hp, degp, b_row, w)


def _t_last(aggp, hp, degp, b_row):
    def body(ap_ref, hp_ref, dp_ref, b_ref, o_ref):
        s = _deg_s(dp_ref[0], dp_ref[1])
        o = (ap_ref[0] + ap_ref[1] + hp_ref[...]) * s + b_ref[...]
        col = lax.broadcasted_iota(jnp.int32, (_BN, _DO), 1)
        mask = col < _NCLS
        om = jnp.where(mask, o, -jnp.inf)
        m = jnp.max(om, axis=1, keepdims=True)
        e = jnp.where(mask, jnp.exp(o - m), 0.0)
        lse = jnp.log(jnp.sum(e, axis=1, keepdims=True))
        r = o - m - lse
        o_ref[...] = r[:, :_NCLS]

    return pl.pallas_call(
        body,
        grid=(_N // _BN,),
        in_specs=[
            pl.BlockSpec((2, _BN, _DO), lambda i: (0, i, 0)),
            pl.BlockSpec((_BN, _DO), lambda i: (i, 0)),
            pl.BlockSpec((2, _BN, _D), lambda i: (0, i, 0)),
            pl.BlockSpec((1, _DO), lambda i: (0, 0)),
        ],
        out_specs=pl.BlockSpec((_BN, _NCLS), lambda i: (i, 0)),
        out_shape=jax.ShapeDtypeStruct((_N, _NCLS), jnp.float32),
    )(aggp, hp, degp, b_row)


def kernel(x, edge_index, W1, b1, W2, b2, W3, b3):
    src = edge_index[0].astype(jnp.int32)
    dst = edge_index[1].astype(jnp.int32)

    npad = _EP - _E
    pad_i = jnp.arange(npad, dtype=jnp.int32)
    src_p = jnp.concatenate([src, pad_i % _N]).reshape(_EP // _K, _K)
    dst_p = jnp.concatenate([dst, _N + (pad_i % (_ACC_N - _N))]).reshape(
        _EP // _K, _K)

    ones128 = jnp.zeros((_K, _D), jnp.float32).at[:, 0].set(1.0)
    z128 = jnp.zeros((_RPT, _D), jnp.float32)

    b1r = b1.reshape(1, _D)
    b2r = b2.reshape(1, _D)
    b3r = jnp.concatenate([b3, jnp.zeros((_DO - _NCLS,), jnp.float32)]
                          ).reshape(1, _DO)
    w3p = jnp.concatenate(
        [W3, jnp.zeros((_D, _DO - _NCLS), jnp.float32)], axis=1)

    degp = _make_deg_kernel()(dst_p, ones128, z128)

    hp1 = _t_first(x, W1, degp)
    agg1 = _make_scatter_kernel(_D)(hp1, src_p, dst_p, z128)
    hp2 = _t_mid(agg1, hp1, degp, b1r, W2, _D)
    agg2 = _make_scatter_kernel(_D)(hp2, src_p, dst_p, z128)
    hp3 = _t_mid(agg2, hp2, degp, b2r, w3p, _DO)
    agg3 = _make_scatter_kernel(_DO)(hp3, src_p, dst_p, z128)
    return _t_last(agg3, hp3, degp, b3r)


# speedup vs baseline: 25.0142x; 1.0103x over previous
import functools

import jax
import jax.numpy as jnp
from jax import lax
from jax.experimental import pallas as pl
from jax.experimental.pallas import tpu as pltpu
from jax.experimental.pallas import tpu_sc as plsc

_N = 10000
_E = 320000
_D = 128
_DO = 128
_NCLS = 40

_NTILES = 32
_K = 128
_PT = 10240
_EP = _NTILES * _PT
_CHUNKS = _PT // _K
_ACC_N = 10112
_RPT = _ACC_N // 16
_BN = 1000


def _sc_mesh():
    return plsc.VectorSubcoreMesh(core_axis_name="c", subcore_axis_name="s")


def _make_deg_kernel():
    @functools.partial(
        pl.kernel,
        mesh=_sc_mesh(),
        out_type=jax.ShapeDtypeStruct((2, _ACC_N, _D), jnp.float32),
        scratch_types=[
            pltpu.VMEM((_CHUNKS, _K), jnp.int32),
            pltpu.VMEM((_K, _D), jnp.float32),
            pltpu.VMEM_SHARED((_ACC_N, _D), jnp.float32),
            pltpu.SemaphoreType.DMA,
        ],
    )
    def deg_kernel(dst_hbm, ones_hbm, zero_hbm, out_hbm, dstv, onesv, acc,
                   sem):
        c = lax.axis_index("c")
        s = lax.axis_index("s")
        t = s * 2 + c
        row0 = s * _RPT
        pltpu.sync_copy(zero_hbm, acc.at[pl.ds(row0, _RPT)])
        pltpu.sync_copy(ones_hbm, onesv)
        pltpu.sync_copy(dst_hbm.at[pl.ds(t * _CHUNKS, _CHUNKS)], dstv)
        plsc.subcore_barrier()

        def _wait_one(carry):
            pltpu.make_async_copy(onesv, acc.at[dstv.at[0]], sem).wait()
            return carry

        _depth = 4

        def body(j, carry):
            pltpu.async_copy(onesv, acc.at[dstv.at[j]], sem, add=True)
            return lax.cond(j >= _depth, _wait_one, lambda carr: carr, carry)

        lax.fori_loop(0, _CHUNKS, body, 0)
        for _ in range(_depth):
            _wait_one(0)
        plsc.subcore_barrier()
        pltpu.sync_copy(acc.at[pl.ds(row0, _RPT)],
                        out_hbm.at[c, pl.ds(row0, _RPT)])

    return deg_kernel


def _make_scatter_kernel(d):
    ph_chunks = _CHUNKS // 2

    @functools.partial(
        pl.kernel,
        mesh=_sc_mesh(),
        out_type=jax.ShapeDtypeStruct((2, _ACC_N, d), jnp.float32),
        scratch_types=[
            pltpu.VMEM((ph_chunks, _K), jnp.int32),
            pltpu.VMEM((ph_chunks, _K), jnp.int32),
            pltpu.VMEM((_K, d), jnp.float32),
            pltpu.VMEM((_K, d), jnp.float32),
            pltpu.VMEM_SHARED((_ACC_N, d), jnp.float32),
            pltpu.SemaphoreType.DMA,
            pltpu.SemaphoreType.DMA,
            pltpu.SemaphoreType.DMA,
        ],
    )
    def scat_kernel(hp_hbm, src_hbm, dst_hbm, zero_hbm, out_hbm,
                    srcv, dstv, rows0, rows1, acc, sem0, sem1, semz):
        c = lax.axis_index("c")
        s = lax.axis_index("s")
        t = s * 2 + c
        row0 = s * _RPT
        pltpu.async_copy(zero_hbm, acc.at[pl.ds(row0, _RPT)], semz)

        def _wait(buf, sem):
            pltpu.make_async_copy(hp_hbm.at[pl.ds(0, _K)], buf, sem).wait()

        first = True
        for phase in range(2):
            base = t * _CHUNKS + phase * ph_chunks
            pltpu.sync_copy(src_hbm.at[pl.ds(base, ph_chunks)], srcv)
            pltpu.sync_copy(dst_hbm.at[pl.ds(base, ph_chunks)], dstv)

            pltpu.async_copy(hp_hbm.at[srcv.at[0]], rows0, sem0)
            if first:
                pltpu.make_async_copy(zero_hbm, acc.at[pl.ds(row0, _RPT)],
                                      semz).wait()
                plsc.subcore_barrier()
                first = False

            def body(i, carry):
                j0 = 2 * i
                j1 = j0 + 1
                pltpu.async_copy(hp_hbm.at[srcv.at[j1]], rows1, sem1)
                _wait(rows0, sem0)
                pltpu.sync_copy(rows0, acc.at[dstv.at[j0]], add=True)
                jn = jnp.minimum(j1 + 1, ph_chunks - 1)
                pltpu.async_copy(hp_hbm.at[srcv.at[jn]], rows0, sem0)
                _wait(rows1, sem1)
                pltpu.sync_copy(rows1, acc.at[dstv.at[j1]], add=True)
                return carry

            lax.fori_loop(0, ph_chunks // 2, body, 0)
            _wait(rows0, sem0)
        plsc.subcore_barrier()
        pltpu.sync_copy(acc.at[pl.ds(row0, _RPT)],
                        out_hbm.at[c, pl.ds(row0, _RPT)])

    return scat_kernel


def _deg_s(dp0, dp1):
    deg = dp0[:, 0:1] + dp1[:, 0:1] + 1.0
    return lax.rsqrt(deg)


def _t_mm(x, w1):
    def body(x_ref, w_ref, o_ref):
        o_ref[...] = jnp.dot(x_ref[...], w_ref[...],
                             preferred_element_type=jnp.float32)

    return pl.pallas_call(
        body,
        grid=(_N // _BN,),
        in_specs=[
            pl.BlockSpec((_BN, _D), lambda i: (i, 0)),
            pl.BlockSpec((_D, _D), lambda i: (0, 0)),
        ],
        out_specs=pl.BlockSpec((_BN, _D), lambda i: (i, 0)),
        out_shape=jax.ShapeDtypeStruct((_N, _D), jnp.float32),
    )(x, w1)


def _t_scale(u, degp):
    def body(u_ref, dp_ref, o_ref):
        s = _deg_s(dp_ref[0], dp_ref[1])
        o_ref[...] = u_ref[...] * s

    return pl.pallas_call(
        body,
        grid=(_N // _BN,),
        in_specs=[
            pl.BlockSpec((_BN, _D), lambda i: (i, 0)),
            pl.BlockSpec((2, _BN, _D), lambda i: (0, i, 0)),
        ],
        out_specs=pl.BlockSpec((_BN, _D), lambda i: (i, 0)),
        out_shape=jax.ShapeDtypeStruct((_N, _D), jnp.float32),
    )(u, degp)


def _t_mid(aggp, hp, degp, b_row, w, dout):
    def body(ap_ref, hp_ref, dp_ref, b_ref, w_ref, o_ref):
        s = _deg_s(dp_ref[0], dp_ref[1])
        h = (ap_ref[0] + ap_ref[1] + hp_ref[...]) * s + b_ref[...]
        h = jnp.maximum(h, 0.0)
        o_ref[...] = jnp.dot(h, w_ref[...],
                             preferred_element_type=jnp.float32) * s

    return pl.pallas_call(
        body,
        grid=(_N // _BN,),
        in_specs=[
            pl.BlockSpec((2, _BN, _D), lambda i: (0, i, 0)),
            pl.BlockSpec((_BN, _D), lambda i: (i, 0)),
            pl.BlockSpec((2, _BN, _D), lambda i: (0, i, 0)),
            pl.BlockSpec((1, _D), lambda i: (0, 0)),
            pl.BlockSpec((_D, dout), lambda i: (0, 0)),
        ],
        out_specs=pl.BlockSpec((_BN, dout), lambda i: (i, 0)),
        out_shape=jax.ShapeDtypeStruct((_N, dout), jnp.float32),
    )(aggp, hp, degp, b_row, w)


def _t_last(aggp, hp, degp, b_row):
    def body(ap_ref, hp_ref, dp_ref, b_ref, o_ref):
        s = _deg_s(dp_ref[0], dp_ref[1])
        o = (ap_ref[0] + ap_ref[1] + hp_ref[...]) * s + b_ref[...]
        col = lax.broadcasted_iota(jnp.int32, (_BN, _DO), 1)
        mask = col < _NCLS
        om = jnp.where(mask, o, -jnp.inf)
        m = jnp.max(om, axis=1, keepdims=True)
        e = jnp.where(mask, jnp.exp(o - m), 0.0)
        lse = jnp.log(jnp.sum(e, axis=1, keepdims=True))
        r = o - m - lse
        o_ref[...] = r[:, :_NCLS]

    return pl.pallas_call(
        body,
        grid=(_N // _BN,),
        in_specs=[
            pl.BlockSpec((2, _BN, _DO), lambda i: (0, i, 0)),
            pl.BlockSpec((_BN, _DO), lambda i: (i, 0)),
            pl.BlockSpec((2, _BN, _D), lambda i: (0, i, 0)),
            pl.BlockSpec((1, _DO), lambda i: (0, 0)),
        ],
        out_specs=pl.BlockSpec((_BN, _NCLS), lambda i: (i, 0)),
        out_shape=jax.ShapeDtypeStruct((_N, _NCLS), jnp.float32),
    )(aggp, hp, degp, b_row)


def kernel(x, edge_index, W1, b1, W2, b2, W3, b3):
    src = edge_index[0].astype(jnp.int32)
    dst = edge_index[1].astype(jnp.int32)

    npad = _EP - _E
    pad_i = jnp.arange(npad, dtype=jnp.int32)
    src_p = jnp.concatenate([src, pad_i % _N]).reshape(_EP // _K, _K)
    dst_p = jnp.concatenate([dst, _N + (pad_i % (_ACC_N - _N))]).reshape(
        _EP // _K, _K)

    ones128 = jnp.zeros((_K, _D), jnp.float32).at[:, 0].set(1.0)
    z128 = jnp.zeros((_RPT, _D), jnp.float32)

    b1r = b1.reshape(1, _D)
    b2r = b2.reshape(1, _D)
    b3r = jnp.concatenate([b3, jnp.zeros((_DO - _NCLS,), jnp.float32)]
                          ).reshape(1, _DO)
    w3p = jnp.concatenate(
        [W3, jnp.zeros((_D, _DO - _NCLS), jnp.float32)], axis=1)

    u1 = _t_mm(x, W1)
    degp = _make_deg_kernel()(dst_p, ones128, z128)

    hp1 = _t_scale(u1, degp)
    agg1 = _make_scatter_kernel(_D)(hp1, src_p, dst_p, z128)
    hp2 = _t_mid(agg1, hp1, degp, b1r, W2, _D)
    agg2 = _make_scatter_kernel(_D)(hp2, src_p, dst_p, z128)
    hp3 = _t_mid(agg2, hp2, degp, b2r, w3p, _DO)
    agg3 = _make_scatter_kernel(_DO)(hp3, src_p, dst_p, z128)
    return _t_last(agg3, hp3, degp, b3r)
